# baseline (speedup 1.0000x reference)
import jax
import jax.numpy as jnp
from jax.experimental import pallas as pl
from jax.experimental.pallas import tpu as pltpu


def _rpn_head_kernel(xf_ref, wall_ref, bc_ref, whc_ref, whb_ref,
                     bhc_ref, bhb_ref, cls_ref, bbox_ref, xcol_ref):
    _, c, h, w = xf_ref.shape
    hw = h * w
    x = xf_ref[0].astype(jnp.bfloat16).reshape(c, hw)
    wpos = jax.lax.broadcasted_iota(jnp.int32, (1, hw), 1) % w
    m_lo = wpos != 0
    m_hi = wpos != (w - 1)
    for dy in (-1, 0, 1):
        for dx in (-1, 0, 1):
            d = w * dy + dx
            if d > 0:
                s = jnp.concatenate(
                    [x[:, d:], jnp.zeros((c, d), jnp.bfloat16)], axis=1)
            elif d < 0:
                s = jnp.concatenate(
                    [jnp.zeros((c, -d), jnp.bfloat16), x[:, :hw + d]], axis=1)
            else:
                s = x
            if dx == -1:
                s = jnp.where(m_lo, s, jnp.bfloat16(0))
            elif dx == 1:
                s = jnp.where(m_hi, s, jnp.bfloat16(0))
            t_idx = 3 * (dy + 1) + (dx + 1)
            xcol_ref[c * t_idx:c * (t_idx + 1), :] = s
    acc = jnp.dot(wall_ref[...], xcol_ref[...],
                  preferred_element_type=jnp.float32)
    t = jnp.maximum(acc + bc_ref[...], 0.0)
    nc = cls_ref.shape[1]
    nb = bbox_ref.shape[1]
    cls = jnp.dot(whc_ref[...], t, preferred_element_type=jnp.float32)
    bbox = jnp.dot(whb_ref[...], t, preferred_element_type=jnp.float32)
    cls_ref[0] = (cls + bhc_ref[...]).reshape(nc, h, w)
    bbox_ref[0] = (bbox + bhb_ref[...]).reshape(nb, h, w)


def kernel(features, W_conv, b_conv, W_cls, b_cls, W_bbox, b_bbox):
    B, C, H, W = features.shape
    k = W_cls.shape[0]
    k4 = W_bbox.shape[0]

    wall = jnp.transpose(W_conv, (0, 2, 3, 1)).reshape(C, 9 * C)
    wall = wall.astype(jnp.bfloat16)
    whc = W_cls[:, :, 0, 0]
    whb = W_bbox[:, :, 0, 0]
    bhc = b_cls.reshape(k, 1)
    bhb = b_bbox.reshape(k4, 1)
    bc = b_conv.reshape(C, 1)

    cls_f, bbox_f = pl.pallas_call(
        _rpn_head_kernel,
        grid=(B,),
        in_specs=[
            pl.BlockSpec((1, C, H, W), lambda b: (b, 0, 0, 0)),
            pl.BlockSpec((C, 9 * C), lambda b: (0, 0)),
            pl.BlockSpec((C, 1), lambda b: (0, 0)),
            pl.BlockSpec((k, C), lambda b: (0, 0)),
            pl.BlockSpec((k4, C), lambda b: (0, 0)),
            pl.BlockSpec((k, 1), lambda b: (0, 0)),
            pl.BlockSpec((k4, 1), lambda b: (0, 0)),
        ],
        out_specs=[
            pl.BlockSpec((1, k, H, W), lambda b: (b, 0, 0, 0)),
            pl.BlockSpec((1, k4, H, W), lambda b: (b, 0, 0, 0)),
        ],
        out_shape=[
            jax.ShapeDtypeStruct((B, k, H, W), jnp.float32),
            jax.ShapeDtypeStruct((B, k4, H, W), jnp.float32),
        ],
        scratch_shapes=[pltpu.VMEM((9 * C, H * W), jnp.bfloat16)],
    )(features, wall, bc, whc, whb, bhc, bhb)

    return (cls_f, bbox_f)


# speedup vs baseline: 1.3635x; 1.3635x over previous
import jax
import jax.numpy as jnp
from jax.experimental import pallas as pl
from jax.experimental.pallas import tpu as pltpu


def _rpn_head_kernel(xf_ref, wall_ref, bc_ref, whc_ref, whb_ref,
                     bhc_ref, bhb_ref, cls_ref, bbox_ref, xcol_ref):
    _, hw, c = xf_ref.shape
    h = cls_ref.shape[2]
    w = cls_ref.shape[3]
    x = xf_ref[0].astype(jnp.bfloat16)

    @pl.when(pl.program_id(0) == 0)
    def _zero_pad_cols():
        for t_idx in range(9):
            xcol_ref[:, 256 * t_idx + c:256 * (t_idx + 1)] = jnp.zeros(
                (hw, 256 - c), jnp.bfloat16)

    qpos = jax.lax.broadcasted_iota(jnp.int32, (hw, 1), 0) % w
    m_lo = qpos != 0
    m_hi = qpos != (w - 1)
    for dy in (-1, 0, 1):
        for dx in (-1, 0, 1):
            d = w * dy + dx
            if d > 0:
                s = jnp.concatenate(
                    [x[d:, :], jnp.zeros((d, c), jnp.bfloat16)], axis=0)
            elif d < 0:
                s = jnp.concatenate(
                    [jnp.zeros((-d, c), jnp.bfloat16), x[:hw + d, :]], axis=0)
            else:
                s = x
            if dx == -1:
                s = jnp.where(m_lo, s, jnp.bfloat16(0))
            elif dx == 1:
                s = jnp.where(m_hi, s, jnp.bfloat16(0))
            t_idx = 3 * (dy + 1) + (dx + 1)
            xcol_ref[:, 256 * t_idx:256 * t_idx + c] = s
    acc = jax.lax.dot_general(
        wall_ref[...], xcol_ref[...],
        dimension_numbers=(((1,), (1,)), ((), ())),
        preferred_element_type=jnp.float32)
    t = jnp.maximum(acc + bc_ref[...], 0.0)
    nc = cls_ref.shape[1]
    nb = bbox_ref.shape[1]
    cls = jnp.dot(whc_ref[...], t, preferred_element_type=jnp.float32)
    bbox = jnp.dot(whb_ref[...], t, preferred_element_type=jnp.float32)
    cls_ref[0] = (cls + bhc_ref[...]).reshape(nc, h, w)
    bbox_ref[0] = (bbox + bhb_ref[...]).reshape(nb, h, w)


def kernel(features, W_conv, b_conv, W_cls, b_cls, W_bbox, b_bbox):
    B, C, H, W = features.shape
    k = W_cls.shape[0]
    k4 = W_bbox.shape[0]
    HW = H * W

    xt = jnp.transpose(features, (0, 2, 3, 1)).reshape(B, HW, C)

    w9 = jnp.transpose(W_conv, (0, 2, 3, 1)).reshape(C, 9, C)
    w9 = jnp.pad(w9, ((0, 0), (0, 0), (0, 256 - C)))
    wall = w9.reshape(C, 9 * 256).astype(jnp.bfloat16)
    whc = W_cls[:, :, 0, 0]
    whb = W_bbox[:, :, 0, 0]
    bhc = b_cls.reshape(k, 1)
    bhb = b_bbox.reshape(k4, 1)
    bc = b_conv.reshape(C, 1)

    cls_f, bbox_f = pl.pallas_call(
        _rpn_head_kernel,
        grid=(B,),
        in_specs=[
            pl.BlockSpec((1, HW, C), lambda b: (b, 0, 0)),
            pl.BlockSpec((C, 9 * 256), lambda b: (0, 0)),
            pl.BlockSpec((C, 1), lambda b: (0, 0)),
            pl.BlockSpec((k, C), lambda b: (0, 0)),
            pl.BlockSpec((k4, C), lambda b: (0, 0)),
            pl.BlockSpec((k, 1), lambda b: (0, 0)),
            pl.BlockSpec((k4, 1), lambda b: (0, 0)),
        ],
        out_specs=[
            pl.BlockSpec((1, k, H, W), lambda b: (b, 0, 0, 0)),
            pl.BlockSpec((1, k4, H, W), lambda b: (b, 0, 0, 0)),
        ],
        out_shape=[
            jax.ShapeDtypeStruct((B, k, H, W), jnp.float32),
            jax.ShapeDtypeStruct((B, k4, H, W), jnp.float32),
        ],
        scratch_shapes=[pltpu.VMEM((HW, 9 * 256), jnp.bfloat16)],
    )(xt, wall, bc, whc, whb, bhc, bhb)

    return (cls_f, bbox_f)


# speedup vs baseline: 1.4720x; 1.0796x over previous
import jax
import jax.numpy as jnp
from jax.experimental import pallas as pl
from jax.experimental.pallas import tpu as pltpu


def _rpn_head_kernel(xf_ref, wall_ref, whc_ref, whb_ref,
                     bhc_ref, bhb_ref, cls_ref, bbox_ref, xpad_ref):
    _, hw, c = xf_ref.shape
    h = cls_ref.shape[2]
    w = cls_ref.shape[3]
    x = xf_ref[0].astype(jnp.bfloat16)

    @pl.when(pl.program_id(0) == 0)
    def _init_scratch():
        xpad_ref[...] = jnp.zeros(xpad_ref.shape, jnp.bfloat16)
        xpad_ref[64:64 + hw, c:c + 1] = jnp.ones((hw, 1), jnp.bfloat16)

    qpos = jax.lax.broadcasted_iota(jnp.int32, (hw, 1), 0) % w
    m_lo = qpos != 0
    m_hi = qpos != (w - 1)
    for dx in (-1, 0, 1):
        if dx == 1:
            s = jnp.concatenate(
                [x[1:, :], jnp.zeros((1, c), jnp.bfloat16)], axis=0)
            s = jnp.where(m_hi, s, jnp.bfloat16(0))
        elif dx == -1:
            s = jnp.concatenate(
                [jnp.zeros((1, c), jnp.bfloat16), x[:hw - 1, :]], axis=0)
            s = jnp.where(m_lo, s, jnp.bfloat16(0))
        else:
            s = x
        base = 256 * (dx + 1)
        xpad_ref[64:64 + hw, base:base + c] = s

    acc = None
    for dy in (-1, 0, 1):
        r0 = 64 + 64 * dy
        p = jax.lax.dot_general(
            wall_ref[:, 768 * (dy + 1):768 * (dy + 2)],
            xpad_ref[r0:r0 + hw, :],
            dimension_numbers=(((1,), (1,)), ((), ())),
            preferred_element_type=jnp.float32)
        acc = p if acc is None else acc + p
    t = jnp.maximum(acc, 0.0)
    nc = cls_ref.shape[1]
    nb = bbox_ref.shape[1]
    cls = jnp.dot(whc_ref[...], t, preferred_element_type=jnp.float32)
    bbox = jnp.dot(whb_ref[...], t, preferred_element_type=jnp.float32)
    cls_ref[0] = (cls + bhc_ref[...]).reshape(nc, h, w)
    bbox_ref[0] = (bbox + bhb_ref[...]).reshape(nb, h, w)


def kernel(features, W_conv, b_conv, W_cls, b_cls, W_bbox, b_bbox):
    B, C, H, W = features.shape
    k = W_cls.shape[0]
    k4 = W_bbox.shape[0]
    HW = H * W

    xt = jnp.transpose(features, (0, 2, 3, 1)).reshape(B, HW, C)

    w9 = jnp.transpose(W_conv, (0, 2, 3, 1)).reshape(C, 9, C)
    w9 = jnp.pad(w9, ((0, 0), (0, 0), (0, 256 - C)))
    w9 = w9.at[:, 3, C].set(b_conv)
    wall = w9.reshape(C, 9 * 256).astype(jnp.bfloat16)
    whc = W_cls[:, :, 0, 0]
    whb = W_bbox[:, :, 0, 0]
    bhc = b_cls.reshape(k, 1)
    bhb = b_bbox.reshape(k4, 1)

    cls_f, bbox_f = pl.pallas_call(
        _rpn_head_kernel,
        grid=(B,),
        in_specs=[
            pl.BlockSpec((1, HW, C), lambda b: (b, 0, 0)),
            pl.BlockSpec((C, 9 * 256), lambda b: (0, 0)),
            pl.BlockSpec((k, C), lambda b: (0, 0)),
            pl.BlockSpec((k4, C), lambda b: (0, 0)),
            pl.BlockSpec((k, 1), lambda b: (0, 0)),
            pl.BlockSpec((k4, 1), lambda b: (0, 0)),
        ],
        out_specs=[
            pl.BlockSpec((1, k, H, W), lambda b: (b, 0, 0, 0)),
            pl.BlockSpec((1, k4, H, W), lambda b: (b, 0, 0, 0)),
        ],
        out_shape=[
            jax.ShapeDtypeStruct((B, k, H, W), jnp.float32),
            jax.ShapeDtypeStruct((B, k4, H, W), jnp.float32),
        ],
        scratch_shapes=[pltpu.VMEM((HW + 128, 768), jnp.bfloat16)],
    )(xt, wall, whc, whb, bhc, bhb)

    return (cls_f, bbox_f)


# speedup vs baseline: 1.4877x; 1.0106x over previous
import jax
import jax.numpy as jnp
from jax.experimental import pallas as pl
from jax.experimental.pallas import tpu as pltpu


def _rpn_head_kernel(xf_ref, wt_ref, bc_ref, whc_ref, whb_ref,
                     bhc_ref, bhb_ref, cls_ref, bbox_ref, xpad_ref,
                     wall_ref):
    _, hw, c = xf_ref.shape
    h = cls_ref.shape[2]
    w = cls_ref.shape[3]
    x = xf_ref[0].astype(jnp.bfloat16)

    @pl.when(pl.program_id(0) == 0)
    def _init_scratch():
        xpad_ref[...] = jnp.zeros(xpad_ref.shape, jnp.bfloat16)
        wall_ref[...] = jnp.zeros(wall_ref.shape, jnp.bfloat16)
        for kh in range(3):
            for kw in range(3):
                t_idx = 3 * kh + kw
                wall_ref[:, 256 * t_idx:256 * t_idx + c] = (
                    wt_ref[kh, kw].astype(jnp.bfloat16))

    qpos = jax.lax.broadcasted_iota(jnp.int32, (hw, 1), 0) % w
    m_lo = qpos != 0
    m_hi = qpos != (w - 1)
    for dx in (-1, 0, 1):
        if dx == 1:
            s = jnp.concatenate(
                [x[1:, :], jnp.zeros((1, c), jnp.bfloat16)], axis=0)
            s = jnp.where(m_hi, s, jnp.bfloat16(0))
        elif dx == -1:
            s = jnp.concatenate(
                [jnp.zeros((1, c), jnp.bfloat16), x[:hw - 1, :]], axis=0)
            s = jnp.where(m_lo, s, jnp.bfloat16(0))
        else:
            s = x
        base = 256 * (dx + 1)
        xpad_ref[64:64 + hw, base:base + c] = s

    acc = None
    for dy in (-1, 0, 1):
        r0 = 64 + 64 * dy
        p = jax.lax.dot_general(
            wall_ref[:, 768 * (dy + 1):768 * (dy + 2)],
            xpad_ref[r0:r0 + hw, :],
            dimension_numbers=(((1,), (1,)), ((), ())),
            preferred_element_type=jnp.float32)
        acc = p if acc is None else acc + p
    t = jnp.maximum(acc + bc_ref[...], 0.0)
    nc = cls_ref.shape[1]
    nb = bbox_ref.shape[1]
    cls = jnp.dot(whc_ref[...], t, preferred_element_type=jnp.float32)
    bbox = jnp.dot(whb_ref[...], t, preferred_element_type=jnp.float32)
    cls_ref[0] = (cls + bhc_ref[...]).reshape(nc, h, w)
    bbox_ref[0] = (bbox + bhb_ref[...]).reshape(nb, h, w)


def kernel(features, W_conv, b_conv, W_cls, b_cls, W_bbox, b_bbox):
    B, C, H, W = features.shape
    k = W_cls.shape[0]
    k4 = W_bbox.shape[0]
    HW = H * W

    xt = jnp.transpose(features, (0, 2, 3, 1)).reshape(B, HW, C)
    wt = jnp.transpose(W_conv, (2, 3, 0, 1))
    bc = b_conv.reshape(C, 1)
    whc = W_cls[:, :, 0, 0]
    whb = W_bbox[:, :, 0, 0]
    bhc = b_cls.reshape(k, 1)
    bhb = b_bbox.reshape(k4, 1)

    cls_f, bbox_f = pl.pallas_call(
        _rpn_head_kernel,
        grid=(B,),
        in_specs=[
            pl.BlockSpec((1, HW, C), lambda b: (b, 0, 0)),
            pl.BlockSpec((3, 3, C, C), lambda b: (0, 0, 0, 0)),
            pl.BlockSpec((C, 1), lambda b: (0, 0)),
            pl.BlockSpec((k, C), lambda b: (0, 0)),
            pl.BlockSpec((k4, C), lambda b: (0, 0)),
            pl.BlockSpec((k, 1), lambda b: (0, 0)),
            pl.BlockSpec((k4, 1), lambda b: (0, 0)),
        ],
        out_specs=[
            pl.BlockSpec((1, k, H, W), lambda b: (b, 0, 0, 0)),
            pl.BlockSpec((1, k4, H, W), lambda b: (b, 0, 0, 0)),
        ],
        out_shape=[
            jax.ShapeDtypeStruct((B, k, H, W), jnp.float32),
            jax.ShapeDtypeStruct((B, k4, H, W), jnp.float32),
        ],
        scratch_shapes=[
            pltpu.VMEM((HW + 128, 768), jnp.bfloat16),
            pltpu.VMEM((C, 9 * 256), jnp.bfloat16),
        ],
    )(xt, wt, bc, whc, whb, bhc, bhb)

    return (cls_f, bbox_f)
